# baseline (speedup 1.0000x reference)
import jax
import jax.numpy as jnp
from jax.experimental import pallas as pl
from jax.experimental.pallas import tpu as pltpu

NT = 4
NH = 12
D = 768
DH = D // NH
K = 8
NF = 2048
N = NT + NF
HK = NH * K
S = NT * HK
R = NH * NT
NEG = -3.0e38


def _fused_kernel(x_ref, wq_ref, wk_ref, wexp_ref, wv_ref, out_ref):
    feat = x_ref[0, NT:, :]
    q_rows = [
        jnp.dot(x_ref[0, t:t + 1, :], wq_ref[t],
                preferred_element_type=jnp.float32)
        for t in range(NT)
    ]
    qm = jnp.concatenate(q_rows, axis=0)
    scale = DH ** -0.5
    lg_rows = []
    for h in range(NH):
        kh = jnp.dot(feat, wk_ref[:, h * DH:(h + 1) * DH],
                     preferred_element_type=jnp.float32)
        qh = qm[:, h * DH:(h + 1) * DH]
        lg_rows.append(jax.lax.dot_general(
            qh, kh, (((1,), (1,)), ((), ())),
            preferred_element_type=jnp.float32) * scale)
    lg = jnp.concatenate(lg_rows, axis=0)
    iota = jax.lax.broadcasted_iota(jnp.int32, (R, NF), 1)
    vals, idxs = [], []
    for _ in range(K):
        m = jnp.max(lg, axis=1, keepdims=True)
        am = jnp.min(jnp.where(lg == m, iota, NF), axis=1, keepdims=True)
        vals.append(m)
        idxs.append(am)
        lg = jnp.where(iota == am, NEG, lg)
    tv = jnp.concatenate(vals, axis=1)
    ti = jnp.concatenate(idxs, axis=1).astype(jnp.float32)
    e = jnp.exp(tv - tv[:, 0:1])
    tvs = e / jnp.sum(e, axis=1, keepdims=True)
    hi = (jax.lax.broadcasted_iota(jnp.int32, (S, R), 0) // K) % NH
    ts = jax.lax.broadcasted_iota(jnp.int32, (S, R), 0) // HK
    rr = jax.lax.broadcasted_iota(jnp.int32, (S, R), 1)
    lsel = (rr == hi * NT + ts).astype(jnp.float32)
    kk = jax.lax.broadcasted_iota(jnp.int32, (S, K), 1)
    ii = jax.lax.broadcasted_iota(jnp.int32, (S, K), 0) % K
    rk = (kk == ii).astype(jnp.float32)
    ti_rows = jnp.dot(lsel, ti, preferred_element_type=jnp.float32,
                      precision=jax.lax.Precision.HIGHEST)
    tw_rows = jnp.dot(lsel, tvs, preferred_element_type=jnp.float32,
                      precision=jax.lax.Precision.HIGHEST)
    idx_col = (jnp.sum(ti_rows * rk, axis=1, keepdims=True)
               + 0.5).astype(jnp.int32)
    w_col = jnp.sum(tw_rows * rk, axis=1, keepdims=True)
    iota_n = jax.lax.broadcasted_iota(jnp.int32, (S, NF), 1)
    g0 = (iota_n == idx_col).astype(jnp.bfloat16)
    a0 = jnp.dot(g0, feat.astype(jnp.bfloat16),
                 preferred_element_type=jnp.float32)
    a = a0 * w_col
    ih = (jax.lax.broadcasted_iota(jnp.int32, (S, D), 0) // K) % NH
    ic = jax.lax.broadcasted_iota(jnp.int32, (S, D), 1) // DH
    am = jnp.where(ih == ic, a, 0.0).astype(jnp.bfloat16)
    p_rows = [
        jnp.dot(am[t * HK:(t + 1) * HK, :], wexp_ref[t].astype(jnp.bfloat16),
                preferred_element_type=jnp.float32)
        for t in range(NT)
    ]
    p = jnp.concatenate(p_rows, axis=0).astype(jnp.bfloat16)
    scat = jax.lax.dot_general(g0, p, (((0,), (0,)), ((), ())),
                               preferred_element_type=jnp.float32)
    sr = jax.lax.broadcasted_iota(jnp.int32, (NT * NH, S), 0)
    si = jax.lax.broadcasted_iota(jnp.int32, (NT * NH, S), 1) // K
    smat = (sr == si).astype(jnp.float32)
    g = jnp.dot(smat, a, preferred_element_type=jnp.float32)
    mv = jnp.dot(g, wv_ref[...], preferred_element_type=jnp.float32)
    er = jax.lax.broadcasted_iota(jnp.int32, (NT * NH, D), 0) % NH
    ec = jax.lax.broadcasted_iota(jnp.int32, (NT * NH, D), 1) // DH
    mvm = jnp.where(er == ec, mv, 0.0)
    tr = jax.lax.broadcasted_iota(jnp.int32, (NT, NT * NH), 0)
    tc = jax.lax.broadcasted_iota(jnp.int32, (NT, NT * NH), 1) // NH
    tsel = (tr == tc).astype(jnp.float32)
    att = jnp.dot(tsel, mvm, preferred_element_type=jnp.float32)
    tok_rows = [
        jnp.dot(att[t:t + 1, :], wexp_ref[t],
                preferred_element_type=jnp.float32)
        for t in range(NT)
    ]
    tok = jnp.concatenate(tok_rows, axis=0)
    out_ref[0] = jnp.concatenate([tok, scat], axis=0)


def kernel(x, Wq, Wkv, Wexp):
    B = x.shape[0]
    wk = Wkv[:, :D]
    wv = Wkv[:, D:]
    return pl.pallas_call(
        _fused_kernel,
        grid=(B,),
        in_specs=[
            pl.BlockSpec((1, N, D), lambda b: (b, 0, 0)),
            pl.BlockSpec((NT, D, D), lambda b: (0, 0, 0)),
            pl.BlockSpec((D, D), lambda b: (0, 0)),
            pl.BlockSpec((NT, D, D), lambda b: (0, 0, 0)),
            pl.BlockSpec((D, D), lambda b: (0, 0)),
        ],
        out_specs=pl.BlockSpec((1, N, D), lambda b: (b, 0, 0)),
        out_shape=jax.ShapeDtypeStruct((B, N, D), jnp.float32),
    )(x, Wq, wk, Wexp, wv)


# speedup vs baseline: 1.5393x; 1.5393x over previous
import jax
import jax.numpy as jnp
from jax.experimental import pallas as pl
from jax.experimental.pallas import tpu as pltpu

NT = 4
NH = 12
D = 768
DH = D // NH
K = 8
NF = 2048
N = NT + NF
HK = NH * K
S = NT * HK
R = NH * NT
NEG = -3.0e38


def _fused_kernel(x_ref, wq_ref, wk_ref, wexp_ref, wv_ref, out_ref):
    feat = x_ref[0, NT:, :]
    q_rows = [
        jnp.dot(x_ref[0, t:t + 1, :], wq_ref[t],
                preferred_element_type=jnp.float32)
        for t in range(NT)
    ]
    qm = jnp.concatenate(q_rows, axis=0)
    scale = DH ** -0.5
    kmat = jnp.dot(feat, wk_ref[...], preferred_element_type=jnp.float32)
    r_i = jax.lax.broadcasted_iota(jnp.int32, (R, NT), 0) % NT
    t_i = jax.lax.broadcasted_iota(jnp.int32, (R, NT), 1)
    texp = (r_i == t_i).astype(jnp.float32)
    qex = jnp.dot(texp, qm, preferred_element_type=jnp.float32)
    rh = jax.lax.broadcasted_iota(jnp.int32, (R, D), 0) // NT
    ch = jax.lax.broadcasted_iota(jnp.int32, (R, D), 1) // DH
    qmask = jnp.where(rh == ch, qex, 0.0)
    lg = jax.lax.dot_general(
        qmask, kmat, (((1,), (1,)), ((), ())),
        preferred_element_type=jnp.float32) * scale
    iota = jax.lax.broadcasted_iota(jnp.int32, (R, NF), 1)
    vals, idxs = [], []
    for _ in range(K):
        m = jnp.max(lg, axis=1, keepdims=True)
        am = jnp.min(jnp.where(lg == m, iota, NF), axis=1, keepdims=True)
        vals.append(m)
        idxs.append(am)
        lg = jnp.where(iota == am, NEG, lg)
    tv = jnp.concatenate(vals, axis=1)
    ti = jnp.concatenate(idxs, axis=1).astype(jnp.float32)
    e = jnp.exp(tv - tv[:, 0:1])
    tvs = e / jnp.sum(e, axis=1, keepdims=True)
    hi = (jax.lax.broadcasted_iota(jnp.int32, (S, R), 0) // K) % NH
    ts = jax.lax.broadcasted_iota(jnp.int32, (S, R), 0) // HK
    rr = jax.lax.broadcasted_iota(jnp.int32, (S, R), 1)
    lsel = (rr == hi * NT + ts).astype(jnp.float32)
    kk = jax.lax.broadcasted_iota(jnp.int32, (S, K), 1)
    ii = jax.lax.broadcasted_iota(jnp.int32, (S, K), 0) % K
    rk = (kk == ii).astype(jnp.float32)
    ti_rows = jnp.dot(lsel, ti, preferred_element_type=jnp.float32,
                      precision=jax.lax.Precision.HIGHEST)
    tw_rows = jnp.dot(lsel, tvs, preferred_element_type=jnp.float32,
                      precision=jax.lax.Precision.HIGHEST)
    idx_col = (jnp.sum(ti_rows * rk, axis=1, keepdims=True)
               + 0.5).astype(jnp.int32)
    w_col = jnp.sum(tw_rows * rk, axis=1, keepdims=True)
    iota_n = jax.lax.broadcasted_iota(jnp.int32, (S, NF), 1)
    g0 = (iota_n == idx_col).astype(jnp.bfloat16)
    a0 = jnp.dot(g0, feat.astype(jnp.bfloat16),
                 preferred_element_type=jnp.float32)
    a = a0 * w_col
    ih = (jax.lax.broadcasted_iota(jnp.int32, (S, D), 0) // K) % NH
    ic = jax.lax.broadcasted_iota(jnp.int32, (S, D), 1) // DH
    am = jnp.where(ih == ic, a, 0.0).astype(jnp.bfloat16)
    p_rows = [
        jnp.dot(am[t * HK:(t + 1) * HK, :], wexp_ref[t].astype(jnp.bfloat16),
                preferred_element_type=jnp.float32)
        for t in range(NT)
    ]
    p = jnp.concatenate(p_rows, axis=0).astype(jnp.bfloat16)
    scat = jax.lax.dot_general(g0, p, (((0,), (0,)), ((), ())),
                               preferred_element_type=jnp.float32)
    sr = jax.lax.broadcasted_iota(jnp.int32, (NT * NH, S), 0)
    si = jax.lax.broadcasted_iota(jnp.int32, (NT * NH, S), 1) // K
    smat = (sr == si).astype(jnp.float32)
    g = jnp.dot(smat, a, preferred_element_type=jnp.float32)
    mv = jnp.dot(g, wv_ref[...], preferred_element_type=jnp.float32)
    er = jax.lax.broadcasted_iota(jnp.int32, (NT * NH, D), 0) % NH
    ec = jax.lax.broadcasted_iota(jnp.int32, (NT * NH, D), 1) // DH
    mvm = jnp.where(er == ec, mv, 0.0)
    tr = jax.lax.broadcasted_iota(jnp.int32, (NT, NT * NH), 0)
    tc = jax.lax.broadcasted_iota(jnp.int32, (NT, NT * NH), 1) // NH
    tsel = (tr == tc).astype(jnp.float32)
    att = jnp.dot(tsel, mvm, preferred_element_type=jnp.float32)
    tok_rows = [
        jnp.dot(att[t:t + 1, :], wexp_ref[t],
                preferred_element_type=jnp.float32)
        for t in range(NT)
    ]
    tok = jnp.concatenate(tok_rows, axis=0)
    out_ref[0] = jnp.concatenate([tok, scat], axis=0)


def kernel(x, Wq, Wkv, Wexp):
    B = x.shape[0]
    wk = Wkv[:, :D]
    wv = Wkv[:, D:]
    return pl.pallas_call(
        _fused_kernel,
        grid=(B,),
        in_specs=[
            pl.BlockSpec((1, N, D), lambda b: (b, 0, 0)),
            pl.BlockSpec((NT, D, D), lambda b: (0, 0, 0)),
            pl.BlockSpec((D, D), lambda b: (0, 0)),
            pl.BlockSpec((NT, D, D), lambda b: (0, 0, 0)),
            pl.BlockSpec((D, D), lambda b: (0, 0)),
        ],
        out_specs=pl.BlockSpec((1, N, D), lambda b: (b, 0, 0)),
        out_shape=jax.ShapeDtypeStruct((B, N, D), jnp.float32),
    )(x, Wq, wk, Wexp, wv)
